# baseline (speedup 1.0000x reference)
import functools

import jax
import jax.numpy as jnp
from jax import lax
from jax.experimental import pallas as pl
from jax.experimental.pallas import tpu as pltpu
from jax.experimental.pallas import tpu_sc as plsc

_VOC = 1000000
_DIM = 32
_B = 16384
_L = 50

_NC = 2
_NS = 16
_NW = _NC * _NS
_BW = _B // _NW
_BPC = 16
_NCH = _BW // _BPC
_CH = _BPC * _L
_NV = _CH // 16

_mesh = plsc.VectorSubcoreMesh(core_axis_name="c", subcore_axis_name="s")


@functools.partial(
    pl.kernel,
    mesh=_mesh,
    compiler_params=pltpu.CompilerParams(
        use_tc_tiling_on_sc=False, needs_layout_passes=False
    ),
    out_type=jax.ShapeDtypeStruct((_B, _L, _DIM), jnp.float32),
    scratch_types=[
        pltpu.VMEM((_L, _BW), jnp.int32),
        [pltpu.VMEM((_CH,), jnp.int32) for _ in range(2)],
        [pltpu.VMEM((_CH, _DIM), jnp.float32) for _ in range(2)],
        pltpu.SemaphoreType.DMA,
        pltpu.SemaphoreType.DMA((2,)),
        pltpu.SemaphoreType.DMA((2,)),
    ],
)
def _emb_lookup(xt_hbm, table_hbm, out_hbm, yblk, idx_bufs, row_bufs, sem_y, sem_g, sem_o):
    c = lax.axis_index("c")
    s = lax.axis_index("s")
    wid = s * _NC + c
    bcol0 = wid * _BW

    def transform(k, slot):
        def body(v, _):
            pos = jnp.arange(16, dtype=jnp.int32) + v * 16
            l = lax.rem(pos, _L)
            bb = lax.div(pos, _L) + k * _BPC
            vals = plsc.load_gather(yblk, [l, bb])
            idx_bufs[slot][pl.ds(pl.multiple_of(v * 16, 16), 16)] = vals
            return 0

        lax.fori_loop(0, _NV, body, 0, unroll=2)

    def gather_copy(slot):
        return pltpu.make_async_copy(
            table_hbm.at[idx_bufs[slot]], row_bufs[slot], sem_g.at[slot]
        )

    def out_copies(k, slot):
        b0 = bcol0 + k * _BPC
        return [
            pltpu.make_async_copy(
                row_bufs[slot].at[pl.ds(i * _L, _L)],
                out_hbm.at[b0 + i],
                sem_o.at[slot],
            )
            for i in range(_BPC)
        ]

    ycp = pltpu.make_async_copy(
        xt_hbm.at[:, pl.ds(bcol0, _BW)], yblk, sem_y
    )
    ycp.start()
    ycp.wait()

    transform(0, 0)
    gather_copy(0).start()
    for k in range(_NCH):
        slot = k % 2
        nxt = 1 - slot
        if k + 1 < _NCH:
            transform(k + 1, nxt)
        gather_copy(slot).wait()
        for cp in out_copies(k, slot):
            cp.start()
        if k + 1 < _NCH:
            if k >= 1:
                for cp in out_copies(k - 1, nxt):
                    cp.wait()
            gather_copy(nxt).start()

    for k in (_NCH - 2, _NCH - 1):
        for cp in out_copies(k, k % 2):
            cp.wait()


def kernel(x, table):
    return _emb_lookup(jnp.swapaxes(x, 0, 1), table)


# speedup vs baseline: 1.0090x; 1.0090x over previous
import functools

import jax
import jax.numpy as jnp
from jax import lax
from jax.experimental import pallas as pl
from jax.experimental.pallas import tpu as pltpu
from jax.experimental.pallas import tpu_sc as plsc

_VOC = 1000000
_DIM = 32
_B = 16384
_L = 50
_NTOT = _B * _L

_NC = 2
_NS = 16
_NW = _NC * _NS
_PER_W = _NTOT // _NW
_CH = 800
_NCH = _PER_W // _CH
_NBUF = 4
_LAG = 2

_mesh = plsc.VectorSubcoreMesh(core_axis_name="c", subcore_axis_name="s")


@functools.partial(
    pl.kernel,
    mesh=_mesh,
    compiler_params=pltpu.CompilerParams(use_tc_tiling_on_sc=False),
    out_type=jax.ShapeDtypeStruct((_B, _L, _DIM), jnp.float32),
    scratch_types=[
        [pltpu.VMEM((_CH,), jnp.int32) for _ in range(_NBUF)],
        [pltpu.VMEM((_CH, _DIM), jnp.float32) for _ in range(_NBUF)],
        pltpu.SemaphoreType.DMA((_NBUF,)),
        pltpu.SemaphoreType.DMA((_NBUF,)),
        pltpu.SemaphoreType.DMA((_NBUF,)),
    ],
)
def _emb_lookup(x_hbm, table_hbm, out_hbm, idx_bufs, row_bufs, sem_i, sem_g, sem_o):
    c = lax.axis_index("c")
    s = lax.axis_index("s")
    wid = s * _NC + c
    _BPC = _CH // _L
    base = wid * _PER_W
    base_b = wid * (_PER_W // _L)

    def idx_copy(k, slot):
        return pltpu.make_async_copy(
            x_hbm.at[pl.ds(base + k * _CH, _CH)], idx_bufs[slot], sem_i.at[slot]
        )

    def gather_copy(slot):
        return pltpu.make_async_copy(
            table_hbm.at[idx_bufs[slot]], row_bufs[slot], sem_g.at[slot]
        )

    def out_copies(k, slot):
        b0 = base_b + k * _BPC
        return [
            pltpu.make_async_copy(
                row_bufs[slot].at[pl.ds(i * _L, _L)],
                out_hbm.at[b0 + i],
                sem_o.at[slot],
            )
            for i in range(_BPC)
        ]

    for b in range(_NBUF):
        idx_copy(b, b).start()

    for k in range(_NCH):
        slot = k % _NBUF
        idx_copy(k, slot).wait()
        if k >= _NBUF:
            for cp in out_copies(k - _NBUF, slot):
                cp.wait()
        gather_copy(slot).start()
        g = k - _LAG
        if g >= 0:
            gs = g % _NBUF
            gather_copy(gs).wait()
            for cp in out_copies(g, gs):
                cp.start()
            if g + _NBUF < _NCH:
                idx_copy(g + _NBUF, gs).start()

    for g in range(_NCH - _LAG, _NCH):
        gs = g % _NBUF
        gather_copy(gs).wait()
        for cp in out_copies(g, gs):
            cp.start()

    for k in range(_NCH - _NBUF, _NCH):
        for cp in out_copies(k, k % _NBUF):
            cp.wait()


def kernel(x, table):
    return _emb_lookup(x.reshape(_NTOT), table)


# speedup vs baseline: 1.0103x; 1.0014x over previous
import functools

import jax
import jax.numpy as jnp
from jax import lax
from jax.experimental import pallas as pl
from jax.experimental.pallas import tpu as pltpu
from jax.experimental.pallas import tpu_sc as plsc

_VOC = 1000000
_DIM = 32
_B = 16384
_L = 50
_NTOT = _B * _L

_NC = 2
_NS = 16
_NW = _NC * _NS
_PER_W = _NTOT // _NW
_CH = 800
_NCH = _PER_W // _CH
_NBUF = 4
_LAG = 2

_mesh = plsc.VectorSubcoreMesh(core_axis_name="c", subcore_axis_name="s")


@functools.partial(
    pl.kernel,
    mesh=_mesh,
    compiler_params=pltpu.CompilerParams(
        use_tc_tiling_on_sc=False, allow_input_fusion=[0]
    ),
    out_type=jax.ShapeDtypeStruct((_B, _L, _DIM), jnp.float32),
    scratch_types=[
        [pltpu.VMEM((_CH,), jnp.int32) for _ in range(_NBUF)],
        [pltpu.VMEM((_CH, _DIM), jnp.float32) for _ in range(_NBUF)],
        pltpu.SemaphoreType.DMA((_NBUF,)),
        pltpu.SemaphoreType.DMA((_NBUF,)),
        pltpu.SemaphoreType.DMA((_NBUF,)),
    ],
)
def _emb_lookup(x_hbm, table_hbm, out_hbm, idx_bufs, row_bufs, sem_i, sem_g, sem_o):
    c = lax.axis_index("c")
    s = lax.axis_index("s")
    wid = s * _NC + c
    _BPC = _CH // _L
    base = wid * _PER_W
    base_b = wid * (_PER_W // _L)

    def idx_copy(k, slot):
        return pltpu.make_async_copy(
            x_hbm.at[pl.ds(base + k * _CH, _CH)], idx_bufs[slot], sem_i.at[slot]
        )

    def gather_copy(slot):
        return pltpu.make_async_copy(
            table_hbm.at[idx_bufs[slot]], row_bufs[slot], sem_g.at[slot]
        )

    def out_copies(k, slot):
        b0 = base_b + k * _BPC
        return [
            pltpu.make_async_copy(
                row_bufs[slot].at[pl.ds(i * _L, _L)],
                out_hbm.at[b0 + i],
                sem_o.at[slot],
            )
            for i in range(_BPC)
        ]

    for b in range(_NBUF):
        idx_copy(b, b).start()

    for k in range(_NCH):
        slot = k % _NBUF
        idx_copy(k, slot).wait()
        if k >= _NBUF:
            for cp in out_copies(k - _NBUF, slot):
                cp.wait()
        gather_copy(slot).start()
        g = k - _LAG
        if g >= 0:
            gs = g % _NBUF
            gather_copy(gs).wait()
            for cp in out_copies(g, gs):
                cp.start()
            if g + _NBUF < _NCH:
                idx_copy(g + _NBUF, gs).start()

    for g in range(_NCH - _LAG, _NCH):
        gs = g % _NBUF
        gather_copy(gs).wait()
        for cp in out_copies(g, gs):
            cp.start()

    for k in range(_NCH - _NBUF, _NCH):
        for cp in out_copies(k, k % _NBUF):
            cp.wait()


def kernel(x, table):
    return _emb_lookup(x.reshape(_NTOT), table)
